# Initial kernel scaffold; baseline (speedup 1.0000x reference)
import jax
import jax.numpy as jnp
from jax.experimental import pallas as pl


def kernel(x, edge_index, edge_attr, y, num_size, entity_table, rel_table, rel_prior, W_src, W_self, W_edge, b):
    raise NotImplementedError("write your pallas kernel here")


# speedup vs baseline: 2.8100x; 2.8100x over previous
import jax
import jax.numpy as jnp
from jax import lax
from jax.experimental import pallas as pl
from jax.experimental.pallas import tpu as pltpu
from jax.experimental.pallas import tpu_sc as plsc

N_NODES = 10000
NPAD = 10240
N_EDGES = 320000
NC, NS, L = 2, 16, 16
NW = NC * NS
CH = 128
CHUNKS_PER_W = 80
EPAD = NW * CHUNKS_PER_W * CH
D = 128
DE = 16
BATCH = 1024
NRELP = 256

_mesh = plsc.VectorSubcoreMesh(
    core_axis_name="c", subcore_axis_name="s", num_cores=NC, num_subcores=NS)
_sc_params = pltpu.CompilerParams(needs_layout_passes=False)


def _zero_vmem(ref, nrows, ncols):
  def body(r, _):
    for k in range(ncols // L):
      ref[r, pl.ds(k * L, L)] = jnp.zeros((L,), jnp.float32)
    return 0
  lax.fori_loop(0, nrows, body, 0)


def _k1_body(table, idx, dstp, attrp, g128,
             h0_out, e_out,
             accE, idx_v, rows_v, didx, aidx, grow, sem, sem2):
  cid = lax.axis_index("c")
  sid = lax.axis_index("s")
  wid = cid * NS + sid
  rows_per_s = NPAD // NS
  _zero_vmem(grow, CH, D)
  for t in range(rows_per_s // CH):
    pltpu.sync_copy(grow, accE.at[pl.ds(sid * rows_per_s + t * CH, CH)])
  plsc.subcore_barrier()

  rows_per_w = NPAD // NW
  for t in range(rows_per_w // 64):
    base = wid * rows_per_w + t * 64
    pltpu.sync_copy(idx.at[pl.ds(base, 64)], idx_v)
    pltpu.async_copy(table.at[idx_v], rows_v, sem).wait()
    pltpu.sync_copy(rows_v, h0_out.at[pl.ds(base, 64)])

  def chunk_body(c, _):
    base = (wid * CHUNKS_PER_W + c) * CH
    pltpu.sync_copy(dstp.at[pl.ds(base, CH)], didx)
    pltpu.sync_copy(attrp.at[pl.ds(base, CH)], aidx)
    pltpu.async_copy(g128.at[aidx], grow, sem2).wait()
    pltpu.sync_copy(grow, accE.at[didx], add=True)
    return 0
  lax.fori_loop(0, CHUNKS_PER_W, chunk_body, 0)
  plsc.subcore_barrier()
  for t in range(rows_per_s // CH):
    r0 = sid * rows_per_s + t * CH
    pltpu.sync_copy(accE.at[pl.ds(r0, CH)], e_out.at[cid, pl.ds(r0, CH)])


def _k1(table, idx, dstp, attrp, g128):
  return pl.kernel(
      _k1_body,
      out_type=(jax.ShapeDtypeStruct((NPAD, D), jnp.float32),
                jax.ShapeDtypeStruct((NC, NPAD, D), jnp.float32)),
      mesh=_mesh,
      scratch_types=[
          pltpu.VMEM_SHARED((NPAD, D), jnp.float32),
          pltpu.VMEM((64,), jnp.int32),
          pltpu.VMEM((64, D), jnp.float32),
          pltpu.VMEM((CH,), jnp.int32),
          pltpu.VMEM((CH,), jnp.int32),
          pltpu.VMEM((CH, D), jnp.float32),
          pltpu.SemaphoreType.DMA,
          pltpu.SemaphoreType.DMA,
      ],
      compiler_params=_sc_params,
  )(table, idx, dstp, attrp, g128)


def _spmm_chunk_loop(h, srcp, dstp, attrp, wid,
                     sidx, didx, aidx, rows, prior_v, prior_c, accP, sem):
  def chunk_body(c, _):
    base = (wid * CHUNKS_PER_W + c) * CH
    pltpu.sync_copy(srcp.at[pl.ds(base, CH)], sidx)
    pltpu.sync_copy(dstp.at[pl.ds(base, CH)], didx)
    pltpu.sync_copy(attrp.at[pl.ds(base, CH)], aidx)
    pltpu.async_copy(h.at[sidx], rows, sem).wait()
    for j in range(CH // L):
      a16 = aidx[pl.ds(j * L, L)]
      prior_c[pl.ds(j * L, L)] = plsc.load_gather(prior_v, [a16])

    def scale_body(e, _):
      ev = jnp.full((L,), e, jnp.int32)
      pe = plsc.load_gather(prior_c, [ev])
      for k in range(D // L):
        rows[e, pl.ds(k * L, L)] = rows[e, pl.ds(k * L, L)] * pe
      return 0
    lax.fori_loop(0, CH, scale_body, 0)
    pltpu.sync_copy(rows, accP.at[didx], add=True)
    return 0
  lax.fori_loop(0, CHUNKS_PER_W, chunk_body, 0)


def _spmm1_body(h, srcp, dstp, attrp, prior_t,
                p_out,
                accP, sidx, didx, aidx, rows, prior_v, prior_c, sem):
  cid = lax.axis_index("c")
  sid = lax.axis_index("s")
  wid = cid * NS + sid
  rows_per_s = NPAD // NS
  _zero_vmem(rows, CH, D)
  for t in range(rows_per_s // CH):
    pltpu.sync_copy(rows, accP.at[pl.ds(sid * rows_per_s + t * CH, CH)])
  plsc.subcore_barrier()
  pltpu.sync_copy(prior_t, prior_v)
  _spmm_chunk_loop(h, srcp, dstp, attrp, wid,
                   sidx, didx, aidx, rows, prior_v, prior_c, accP, sem)
  plsc.subcore_barrier()
  for t in range(rows_per_s // CH):
    r0 = sid * rows_per_s + t * CH
    pltpu.sync_copy(accP.at[pl.ds(r0, CH)], p_out.at[cid, pl.ds(r0, CH)])


def _spmm1(h, srcp, dstp, attrp, prior_t):
  return pl.kernel(
      _spmm1_body,
      out_type=jax.ShapeDtypeStruct((NC, NPAD, D), jnp.float32),
      mesh=_mesh,
      scratch_types=[
          pltpu.VMEM_SHARED((NPAD, D), jnp.float32),
          pltpu.VMEM((CH,), jnp.int32),
          pltpu.VMEM((CH,), jnp.int32),
          pltpu.VMEM((CH,), jnp.int32),
          pltpu.VMEM((CH, D), jnp.float32),
          pltpu.VMEM((NRELP,), jnp.float32),
          pltpu.VMEM((CH,), jnp.float32),
          pltpu.SemaphoreType.DMA,
      ],
      compiler_params=_sc_params,
  )(h, srcp, dstp, attrp, prior_t)


def _spmm2_body(h, srcp, dstp, attrp, prior_t, tix, t2,
                pg_out, hg_out, tg_out,
                accP, sidx, didx, aidx, rows, prior_v, prior_c, tbuf, tbuf2,
                grows, sem, sem2):
  cid = lax.axis_index("c")
  sid = lax.axis_index("s")
  wid = cid * NS + sid
  rows_per_s = NPAD // NS
  _zero_vmem(rows, CH, D)
  for t in range(rows_per_s // CH):
    pltpu.sync_copy(rows, accP.at[pl.ds(sid * rows_per_s + t * CH, CH)])
  plsc.subcore_barrier()
  pltpu.sync_copy(prior_t, prior_v)
  _spmm_chunk_loop(h, srcp, dstp, attrp, wid,
                   sidx, didx, aidx, rows, prior_v, prior_c, accP, sem)
  plsc.subcore_barrier()

  rb = BATCH // NS
  pltpu.sync_copy(tix.at[pl.ds(sid * rb, rb)], tbuf)
  pltpu.async_copy(accP.at[tbuf], grows, sem).wait()
  pltpu.sync_copy(grows, pg_out.at[cid, pl.ds(sid * rb, rb)])
  rb2 = BATCH // NW
  pltpu.sync_copy(tix.at[pl.ds(wid * rb2, rb2)], tbuf2)
  pltpu.async_copy(h.at[tbuf2], grows.at[pl.ds(0, rb2)], sem).wait()
  pltpu.sync_copy(grows.at[pl.ds(0, rb2)], hg_out.at[pl.ds(wid * rb2, rb2)])
  pltpu.async_copy(t2.at[tbuf2], grows.at[pl.ds(0, rb2)], sem2).wait()
  pltpu.sync_copy(grows.at[pl.ds(0, rb2)], tg_out.at[pl.ds(wid * rb2, rb2)])


def _spmm2(h, srcp, dstp, attrp, prior_t, tix, t2):
  return pl.kernel(
      _spmm2_body,
      out_type=(jax.ShapeDtypeStruct((NC, BATCH, D), jnp.float32),
                jax.ShapeDtypeStruct((BATCH, D), jnp.float32),
                jax.ShapeDtypeStruct((BATCH, D), jnp.float32)),
      mesh=_mesh,
      scratch_types=[
          pltpu.VMEM_SHARED((NPAD, D), jnp.float32),
          pltpu.VMEM((CH,), jnp.int32),
          pltpu.VMEM((CH,), jnp.int32),
          pltpu.VMEM((CH,), jnp.int32),
          pltpu.VMEM((CH, D), jnp.float32),
          pltpu.VMEM((NRELP,), jnp.float32),
          pltpu.VMEM((CH,), jnp.float32),
          pltpu.VMEM((BATCH // NS,), jnp.int32),
          pltpu.VMEM((BATCH // NW,), jnp.int32),
          pltpu.VMEM((BATCH // NS, D), jnp.float32),
          pltpu.SemaphoreType.DMA,
          pltpu.SemaphoreType.DMA,
      ],
      compiler_params=_sc_params,
  )(h, srcp, dstp, attrp, prior_t, tix, t2)


_BLK = 1024


def _layer1_tc_body(p_ref, e_ref, h0_ref, ws_ref, we0_ref, we1_ref, wf_ref,
                    b0_ref, b1_ref, h1_ref, t2_ref):
  ps = p_ref[0] + p_ref[1]
  es = e_ref[0] + e_ref[1]
  acc = jnp.dot(ps, ws_ref[...], preferred_element_type=jnp.float32)
  acc += jnp.dot(es, we0_ref[...], preferred_element_type=jnp.float32)
  acc += jnp.dot(h0_ref[...], wf_ref[...], preferred_element_type=jnp.float32)
  h1_ref[...] = jnp.maximum(acc + b0_ref[...], 0.0)
  t2_ref[...] = jnp.dot(es, we1_ref[...],
                        preferred_element_type=jnp.float32) + b1_ref[...]


def _layer1_tc(p, e, h0, ws0, we0p, we1p, wf0, b0, b1):
  nblk = NPAD // _BLK
  return pl.pallas_call(
      _layer1_tc_body,
      grid=(nblk,),
      in_specs=[
          pl.BlockSpec((NC, _BLK, D), lambda i: (0, i, 0)),
          pl.BlockSpec((NC, _BLK, D), lambda i: (0, i, 0)),
          pl.BlockSpec((_BLK, D), lambda i: (i, 0)),
          pl.BlockSpec((D, D), lambda i: (0, 0)),
          pl.BlockSpec((D, D), lambda i: (0, 0)),
          pl.BlockSpec((D, D), lambda i: (0, 0)),
          pl.BlockSpec((D, D), lambda i: (0, 0)),
          pl.BlockSpec((1, D), lambda i: (0, 0)),
          pl.BlockSpec((1, D), lambda i: (0, 0)),
      ],
      out_specs=[
          pl.BlockSpec((_BLK, D), lambda i: (i, 0)),
          pl.BlockSpec((_BLK, D), lambda i: (i, 0)),
      ],
      out_shape=[
          jax.ShapeDtypeStruct((NPAD, D), jnp.float32),
          jax.ShapeDtypeStruct((NPAD, D), jnp.float32),
      ],
  )(p, e, h0, ws0, we0p, we1p, wf0, b0, b1)


def _final_tc_body(pg_ref, hg_ref, tg_ref, ws_ref, wf_ref, out_ref):
  ps = pg_ref[0] + pg_ref[1]
  acc = jnp.dot(ps, ws_ref[...], preferred_element_type=jnp.float32)
  acc += jnp.dot(hg_ref[...], wf_ref[...], preferred_element_type=jnp.float32)
  out_ref[...] = jnp.maximum(acc + tg_ref[...], 0.0)


def _final_tc(pg, hg, tg, ws1, wf1):
  return pl.pallas_call(
      _final_tc_body,
      out_shape=jax.ShapeDtypeStruct((BATCH, D), jnp.float32),
  )(pg, hg, tg, ws1, wf1)


def kernel(x, edge_index, edge_attr, y, num_size, entity_table, rel_table,
           rel_prior, W_src, W_self, W_edge, b):
  x = x.astype(jnp.int32)
  src = edge_index[0].astype(jnp.int32)
  dst = edge_index[1].astype(jnp.int32)
  attr = edge_attr.astype(jnp.int32)
  y = y.astype(jnp.int32)
  num_size = num_size.astype(jnp.int32)

  x_pad = jnp.pad(x, (0, NPAD - N_NODES))
  src_p = jnp.pad(src, (0, EPAD - N_EDGES))
  dst_p = jnp.pad(dst, (0, EPAD - N_EDGES), constant_values=NPAD - CH)
  attr_p = jnp.pad(attr, (0, EPAD - N_EDGES), constant_values=NRELP - L)
  nrel = rel_prior.shape[0]
  prior_flat = jnp.pad(rel_prior[:, 0], (0, NRELP - nrel))
  g = rel_table * rel_prior
  g128 = jnp.pad(g, ((0, NRELP - nrel), (0, D - DE)))
  g128 = g128.at[NRELP - L].set(0.0)
  we0p = jnp.pad(W_edge[0], ((0, D - DE), (0, 0)))
  we1p = jnp.pad(W_edge[1], ((0, D - DE), (0, 0)))

  offsets = jnp.concatenate(
      [jnp.zeros((1,), jnp.int32), jnp.cumsum(num_size)[:-1]])
  true_idx = (offsets + y).astype(jnp.int32)

  b2 = b.reshape(2, 1, D)

  h0, eagg = _k1(entity_table, x_pad, dst_p, attr_p, g128)
  p1 = _spmm1(h0, src_p, dst_p, attr_p, prior_flat)
  h1, t2 = _layer1_tc(p1, eagg, h0, W_src[0], we0p, we1p,
                      W_self[0], b2[0], b2[1])
  pg, hg, tg = _spmm2(h1, src_p, dst_p, attr_p, prior_flat, true_idx, t2)
  return _final_tc(pg, hg, tg, W_src[1], W_self[1])


# speedup vs baseline: 5.5257x; 1.9664x over previous
import jax
import jax.numpy as jnp
from jax import lax
from jax.experimental import pallas as pl
from jax.experimental.pallas import tpu as pltpu
from jax.experimental.pallas import tpu_sc as plsc

N_NODES = 10000
NPAD = 10240
N_EDGES = 320000
NC, NS, L = 2, 16, 16
NW = NC * NS
CH = 128
NCHW = 80
EPAD = NW * NCHW * CH
D = 128
DE = 16
BATCH = 1024
NRELP = 256

_mesh = plsc.VectorSubcoreMesh(
    core_axis_name="c", subcore_axis_name="s", num_cores=NC, num_subcores=NS)
_sc_params = pltpu.CompilerParams(needs_layout_passes=False)


def _zero_vmem(ref, nrows, ncols):
  def body(r, _):
    for k in range(ncols // L):
      ref[r, pl.ds(k * L, L)] = jnp.zeros((L,), jnp.float32)
    return 0
  lax.fori_loop(0, nrows, body, 0)


def _k1_body(table, idx, edata, g_cm,
             h0_out, e_out,
             accE, x_v, g_v, grow0, grow1, ed0, ed1, ed2, ed3,
             sem, esem0, esem1, ssem0, ssem1):
  cid = lax.axis_index("c")
  sid = lax.axis_index("s")
  wid = cid * NS + sid
  rows_per_s = NPAD // NS
  rows_per_w = NPAD // NW
  ed = (ed0, ed1, ed2, ed3)
  grows = (grow0, grow1)
  esems = (esem0, esem1)
  ssems = (ssem0, ssem1)

  pltpu.sync_copy(idx.at[pl.ds(wid * rows_per_w, rows_per_w)], x_v)
  pltpu.async_copy(table.at[x_v.at[pl.ds(0, CH)]], grow0, sem)
  pltpu.async_copy(table.at[x_v.at[pl.ds(CH, CH)]], grow1, sem)
  pltpu.make_async_copy(table.at[x_v.at[pl.ds(0, CH)]], grow0, sem).wait()
  pltpu.sync_copy(grow0, h0_out.at[pl.ds(wid * rows_per_w, CH)])
  pltpu.async_copy(table.at[x_v.at[pl.ds(2 * CH, 64)]],
                   grow0.at[pl.ds(0, 64)], sem)
  pltpu.make_async_copy(table.at[x_v.at[pl.ds(CH, CH)]], grow1, sem).wait()
  pltpu.sync_copy(grow1, h0_out.at[pl.ds(wid * rows_per_w + CH, CH)])
  pltpu.make_async_copy(table.at[x_v.at[pl.ds(2 * CH, 64)]],
                        grow0.at[pl.ds(0, 64)], sem).wait()
  pltpu.sync_copy(grow0.at[pl.ds(0, 64)],
                  h0_out.at[pl.ds(wid * rows_per_w + 2 * CH, 64)])

  _zero_vmem(grow0, CH, D)
  _zero_vmem(grow1, CH, D)
  for t in range(rows_per_s // CH):
    pltpu.sync_copy(grow0, accE.at[pl.ds(sid * rows_per_s + t * CH, CH)])
  plsc.subcore_barrier()

  pltpu.sync_copy(g_cm, g_v)
  riota = lax.iota(jnp.int32, L)

  def build(grow, edc):
    for j in range(CH // L):
      a16 = edc[2, pl.ds(j * L, L)]
      rr = riota + j * L
      for col in range(DE):
        val = plsc.load_gather(g_v, [a16 + col * NRELP])
        plsc.store_scatter(grow, [rr, jnp.full((L,), col, jnp.int32)], val)

  def idxload(c, slot, sem_):
    pltpu.async_copy(edata.at[pl.ds((wid * NCHW + c) * 8, 8)], ed[slot], sem_)

  def ewait(slot, sem_):
    pltpu.make_async_copy(edata.at[pl.ds(0, 8)], ed[slot], sem_).wait()

  def swait(grow, sem_):
    pltpu.make_async_copy(grow, accE.at[ed0.at[1]], sem_).wait()

  pltpu.sync_copy(edata.at[pl.ds((wid * NCHW) * 8, 8)], ed0)
  pltpu.sync_copy(edata.at[pl.ds((wid * NCHW + 1) * 8, 8)], ed1)

  def body(i, _):
    for k in range(4):
      c = 4 * i + k
      b = k % 2
      if k < 2:
        @pl.when(i > 0)
        def _():
          swait(grows[b], ssems[b])
          ewait(k, esems[b])
      else:
        swait(grows[b], ssems[b])
        ewait(k, esems[b])
      if k < 2:
        idxload(c + 2, (k + 2) % 4, esems[b])
      else:
        @pl.when(i < NCHW // 4 - 1)
        def _():
          idxload(c + 2, (k + 2) % 4, esems[b])
      build(grows[b], ed[k])
      pltpu.async_copy(grows[b], accE.at[ed[k].at[1]], ssems[b], add=True)
    return 0
  lax.fori_loop(0, NCHW // 4, body, 0)
  swait(grow0, ssem0)
  swait(grow1, ssem1)
  plsc.subcore_barrier()
  for t in range(rows_per_s // CH):
    r0 = sid * rows_per_s + t * CH
    pltpu.sync_copy(accE.at[pl.ds(r0, CH)], e_out.at[cid, pl.ds(r0, CH)])


def _k1(table, idx, edata, g_cm):
  return pl.kernel(
      _k1_body,
      out_type=(jax.ShapeDtypeStruct((NPAD, D), jnp.float32),
                jax.ShapeDtypeStruct((NC, NPAD, D), jnp.float32)),
      mesh=_mesh,
      scratch_types=[
          pltpu.VMEM_SHARED((NPAD, D), jnp.float32),
          pltpu.VMEM((NPAD // NW,), jnp.int32),
          pltpu.VMEM((DE * NRELP,), jnp.float32),
          pltpu.VMEM((CH, D), jnp.float32),
          pltpu.VMEM((CH, D), jnp.float32),
          pltpu.VMEM((8, CH), jnp.int32),
          pltpu.VMEM((8, CH), jnp.int32),
          pltpu.VMEM((8, CH), jnp.int32),
          pltpu.VMEM((8, CH), jnp.int32),
          pltpu.SemaphoreType.DMA,
          pltpu.SemaphoreType.DMA,
          pltpu.SemaphoreType.DMA,
          pltpu.SemaphoreType.DMA,
          pltpu.SemaphoreType.DMA,
      ],
      compiler_params=_sc_params,
  )(table, idx, edata, g_cm)


def _scale_rows(edc, rows, prior_v, prior_c):
  for j in range(CH // L):
    a16 = edc[2, pl.ds(j * L, L)]
    prior_c[pl.ds(j * L, L)] = plsc.load_gather(prior_v, [a16])

  def sb(e2, _):
    for dd in range(2):
      e = 2 * e2 + dd
      ev = jnp.full((L,), e, jnp.int32)
      pe = plsc.load_gather(prior_c, [ev])
      for k in range(D // L):
        rows[e, pl.ds(k * L, L)] = rows[e, pl.ds(k * L, L)] * pe
    return 0
  lax.fori_loop(0, CH // 2, sb, 0)


def _spmm_loop(h, edata, wid, ed, rows, prior_v, prior_c, accP, esems, gsems,
               ssems):
  def idxload(c, slot, sem_):
    pltpu.async_copy(edata.at[pl.ds((wid * NCHW + c) * 8, 8)], ed[slot], sem_)

  def ewait(slot, sem_):
    pltpu.make_async_copy(edata.at[pl.ds(0, 8)], ed[slot], sem_).wait()

  def gwait(rowsb, sem_):
    pltpu.make_async_copy(h.at[ed[0].at[0]], rowsb, sem_).wait()

  def swait(rowsb, sem_):
    pltpu.make_async_copy(rowsb, accP.at[ed[0].at[1]], sem_).wait()

  pltpu.sync_copy(edata.at[pl.ds((wid * NCHW) * 8, 8)], ed[0])
  pltpu.sync_copy(edata.at[pl.ds((wid * NCHW + 1) * 8, 8)], ed[1])
  pltpu.async_copy(h.at[ed[0].at[0]], rows[0], gsems[0])

  def body(i, _):
    for k in range(4):
      c = 4 * i + k
      b = k % 2
      nb = 1 - b
      if k == 0:
        @pl.when(i > 0)
        def _():
          swait(rows[nb], ssems[nb])
      else:
        swait(rows[nb], ssems[nb])
      if k < 3:
        if k >= 1:
          ewait((k + 1) % 4, esems[nb])
        else:
          @pl.when(i > 0)
          def _():
            ewait(1, esems[nb])
        pltpu.async_copy(h.at[ed[(k + 1) % 4].at[0]], rows[nb], gsems[nb])
      else:
        @pl.when(i < NCHW // 4 - 1)
        def _():
          ewait(0, esems[nb])
          pltpu.async_copy(h.at[ed[0].at[0]], rows[nb], gsems[nb])
      if k < 2:
        idxload(c + 2, (k + 2) % 4, esems[b])
      else:
        @pl.when(i < NCHW // 4 - 1)
        def _():
          idxload(c + 2, (k + 2) % 4, esems[b])
      gwait(rows[b], gsems[b])
      _scale_rows(ed[k], rows[b], prior_v, prior_c)
      pltpu.async_copy(rows[b], accP.at[ed[k].at[1]], ssems[b], add=True)
    return 0
  lax.fori_loop(0, NCHW // 4, body, 0)
  swait(rows[1], ssems[1])


_SPMM_SCRATCH = [
    pltpu.VMEM_SHARED((NPAD, D), jnp.float32),
    pltpu.VMEM((CH, D), jnp.float32),
    pltpu.VMEM((CH, D), jnp.float32),
    pltpu.VMEM((8, CH), jnp.int32),
    pltpu.VMEM((8, CH), jnp.int32),
    pltpu.VMEM((8, CH), jnp.int32),
    pltpu.VMEM((8, CH), jnp.int32),
    pltpu.VMEM((NRELP,), jnp.float32),
    pltpu.VMEM((CH,), jnp.float32),
    pltpu.SemaphoreType.DMA,
    pltpu.SemaphoreType.DMA,
    pltpu.SemaphoreType.DMA,
    pltpu.SemaphoreType.DMA,
    pltpu.SemaphoreType.DMA,
    pltpu.SemaphoreType.DMA,
]


def _spmm_prelude(accP, rows0, sid):
  rows_per_s = NPAD // NS
  _zero_vmem(rows0, CH, D)
  for t in range(rows_per_s // CH):
    pltpu.sync_copy(rows0, accP.at[pl.ds(sid * rows_per_s + t * CH, CH)])
  plsc.subcore_barrier()


def _spmm1_body(h, edata, prior_t,
                p_out,
                accP, rows0, rows1, ed0, ed1, ed2, ed3, prior_v, prior_c,
                esem0, esem1, gsem0, gsem1, ssem0, ssem1):
  cid = lax.axis_index("c")
  sid = lax.axis_index("s")
  wid = cid * NS + sid
  rows_per_s = NPAD // NS
  _spmm_prelude(accP, rows0, sid)
  pltpu.sync_copy(prior_t, prior_v)
  _spmm_loop(h, edata, wid, (ed0, ed1, ed2, ed3), (rows0, rows1),
             prior_v, prior_c, accP, (esem0, esem1), (gsem0, gsem1),
             (ssem0, ssem1))
  plsc.subcore_barrier()
  for t in range(rows_per_s // CH):
    r0 = sid * rows_per_s + t * CH
    pltpu.sync_copy(accP.at[pl.ds(r0, CH)], p_out.at[cid, pl.ds(r0, CH)])


def _spmm1(h, edata, prior_t):
  return pl.kernel(
      _spmm1_body,
      out_type=jax.ShapeDtypeStruct((NC, NPAD, D), jnp.float32),
      mesh=_mesh,
      scratch_types=list(_SPMM_SCRATCH),
      compiler_params=_sc_params,
  )(h, edata, prior_t)


def _spmm2_body(h, edata, prior_t, tix, t2,
                pg_out, hg_out, tg_out,
                accP, rows0, rows1, ed0, ed1, ed2, ed3, prior_v, prior_c,
                esem0, esem1, gsem0, gsem1, ssem0, ssem1,
                tbuf, tbuf2, sem2):
  cid = lax.axis_index("c")
  sid = lax.axis_index("s")
  wid = cid * NS + sid
  _spmm_prelude(accP, rows0, sid)
  pltpu.sync_copy(prior_t, prior_v)
  _spmm_loop(h, edata, wid, (ed0, ed1, ed2, ed3), (rows0, rows1),
             prior_v, prior_c, accP, (esem0, esem1), (gsem0, gsem1),
             (ssem0, ssem1))
  plsc.subcore_barrier()

  rb = BATCH // NS
  grows = rows0.at[pl.ds(0, rb)]
  pltpu.sync_copy(tix.at[pl.ds(sid * rb, rb)], tbuf)
  pltpu.async_copy(accP.at[tbuf], grows, sem2).wait()
  pltpu.sync_copy(grows, pg_out.at[cid, pl.ds(sid * rb, rb)])
  rb2 = BATCH // NW
  grows2 = rows1.at[pl.ds(0, rb2)]
  pltpu.sync_copy(tix.at[pl.ds(wid * rb2, rb2)], tbuf2)
  pltpu.async_copy(h.at[tbuf2], grows2, sem2).wait()
  pltpu.sync_copy(grows2, hg_out.at[pl.ds(wid * rb2, rb2)])
  pltpu.async_copy(t2.at[tbuf2], grows2, sem2).wait()
  pltpu.sync_copy(grows2, tg_out.at[pl.ds(wid * rb2, rb2)])


def _spmm2(h, edata, prior_t, tix, t2):
  return pl.kernel(
      _spmm2_body,
      out_type=(jax.ShapeDtypeStruct((NC, BATCH, D), jnp.float32),
                jax.ShapeDtypeStruct((BATCH, D), jnp.float32),
                jax.ShapeDtypeStruct((BATCH, D), jnp.float32)),
      mesh=_mesh,
      scratch_types=list(_SPMM_SCRATCH) + [
          pltpu.VMEM((BATCH // NS,), jnp.int32),
          pltpu.VMEM((BATCH // NW,), jnp.int32),
          pltpu.SemaphoreType.DMA,
      ],
      compiler_params=_sc_params,
  )(h, edata, prior_t, tix, t2)


_BLK = 1024


def _layer1_tc_body(p_ref, e_ref, h0_ref, ws_ref, we0_ref, we1_ref, wf_ref,
                    b0_ref, b1_ref, h1_ref, t2_ref):
  ps = p_ref[0] + p_ref[1]
  es = e_ref[0] + e_ref[1]
  acc = jnp.dot(ps, ws_ref[...], preferred_element_type=jnp.float32)
  acc += jnp.dot(es, we0_ref[...], preferred_element_type=jnp.float32)
  acc += jnp.dot(h0_ref[...], wf_ref[...], preferred_element_type=jnp.float32)
  h1_ref[...] = jnp.maximum(acc + b0_ref[...], 0.0)
  t2_ref[...] = jnp.dot(es, we1_ref[...],
                        preferred_element_type=jnp.float32) + b1_ref[...]


def _layer1_tc(p, e, h0, ws0, we0p, we1p, wf0, b0, b1):
  nblk = NPAD // _BLK
  return pl.pallas_call(
      _layer1_tc_body,
      grid=(nblk,),
      in_specs=[
          pl.BlockSpec((NC, _BLK, D), lambda i: (0, i, 0)),
          pl.BlockSpec((NC, _BLK, D), lambda i: (0, i, 0)),
          pl.BlockSpec((_BLK, D), lambda i: (i, 0)),
          pl.BlockSpec((D, D), lambda i: (0, 0)),
          pl.BlockSpec((D, D), lambda i: (0, 0)),
          pl.BlockSpec((D, D), lambda i: (0, 0)),
          pl.BlockSpec((D, D), lambda i: (0, 0)),
          pl.BlockSpec((1, D), lambda i: (0, 0)),
          pl.BlockSpec((1, D), lambda i: (0, 0)),
      ],
      out_specs=[
          pl.BlockSpec((_BLK, D), lambda i: (i, 0)),
          pl.BlockSpec((_BLK, D), lambda i: (i, 0)),
      ],
      out_shape=[
          jax.ShapeDtypeStruct((NPAD, D), jnp.float32),
          jax.ShapeDtypeStruct((NPAD, D), jnp.float32),
      ],
  )(p, e, h0, ws0, we0p, we1p, wf0, b0, b1)


def _final_tc_body(pg_ref, hg_ref, tg_ref, ws_ref, wf_ref, out_ref):
  ps = pg_ref[0] + pg_ref[1]
  acc = jnp.dot(ps, ws_ref[...], preferred_element_type=jnp.float32)
  acc += jnp.dot(hg_ref[...], wf_ref[...], preferred_element_type=jnp.float32)
  out_ref[...] = jnp.maximum(acc + tg_ref[...], 0.0)


def _final_tc(pg, hg, tg, ws1, wf1):
  return pl.pallas_call(
      _final_tc_body,
      out_shape=jax.ShapeDtypeStruct((BATCH, D), jnp.float32),
  )(pg, hg, tg, ws1, wf1)


def kernel(x, edge_index, edge_attr, y, num_size, entity_table, rel_table,
           rel_prior, W_src, W_self, W_edge, b):
  x = x.astype(jnp.int32)
  src = edge_index[0].astype(jnp.int32)
  dst = edge_index[1].astype(jnp.int32)
  attr = edge_attr.astype(jnp.int32)
  y = y.astype(jnp.int32)
  num_size = num_size.astype(jnp.int32)

  x_pad = jnp.pad(x, (0, NPAD - N_NODES))
  src_p = jnp.pad(src, (0, EPAD - N_EDGES))
  dst_p = jnp.pad(dst, (0, EPAD - N_EDGES), constant_values=NPAD - CH)
  attr_p = jnp.pad(attr, (0, EPAD - N_EDGES), constant_values=NRELP - L)
  nrel = rel_prior.shape[0]
  prior_flat = jnp.pad(rel_prior[:, 0], (0, NRELP - nrel))
  g = rel_table * rel_prior
  g_pad = jnp.pad(g, ((0, NRELP - nrel), (0, 0)))
  g_cm = g_pad.T.reshape(-1)

  ed3 = jnp.stack([src_p, dst_p, attr_p])
  edata = jnp.pad(ed3.reshape(3, NW, NCHW, CH).transpose(1, 2, 0, 3),
                  ((0, 0), (0, 0), (0, 5), (0, 0))
                  ).reshape(NW * NCHW * 8, CH)

  we0p = jnp.pad(W_edge[0], ((0, D - DE), (0, 0)))
  we1p = jnp.pad(W_edge[1], ((0, D - DE), (0, 0)))

  offsets = jnp.concatenate(
      [jnp.zeros((1,), jnp.int32), jnp.cumsum(num_size)[:-1]])
  true_idx = (offsets + y).astype(jnp.int32)

  b2 = b.reshape(2, 1, D)

  h0, eagg = _k1(entity_table, x_pad, edata, g_cm)
  p1 = _spmm1(h0, edata, prior_flat)
  h1, t2 = _layer1_tc(p1, eagg, h0, W_src[0], we0p, we1p,
                      W_self[0], b2[0], b2[1])
  pg, hg, tg = _spmm2(h1, edata, prior_flat, true_idx, t2)
  return _final_tc(pg, hg, tg, W_src[1], W_self[1])


# speedup vs baseline: 5.5759x; 1.0091x over previous
import jax
import jax.numpy as jnp
from jax import lax
from jax.experimental import pallas as pl
from jax.experimental.pallas import tpu as pltpu
from jax.experimental.pallas import tpu_sc as plsc

N_NODES = 10000
NPAD = 10240
N_EDGES = 320000
NC, NS, L = 2, 16, 16
NW = NC * NS
CH = 128
NCHW = 80
EPAD = NW * NCHW * CH
D = 128
DE = 16
BATCH = 1024
NRELP = 256

_mesh = plsc.VectorSubcoreMesh(
    core_axis_name="c", subcore_axis_name="s", num_cores=NC, num_subcores=NS)
_sc_params = pltpu.CompilerParams(needs_layout_passes=False)


def _zero_vmem(ref, nrows, ncols):
  def body(r, _):
    for k in range(ncols // L):
      ref[r, pl.ds(k * L, L)] = jnp.zeros((L,), jnp.float32)
    return 0
  lax.fori_loop(0, nrows, body, 0)


def _k1_body(table, idx, edata, g_cm,
             h0_out, e_out,
             accE, x_v, g_v, grow0, grow1, ed0, ed1, ed2, ed3,
             sem, esem0, esem1, ssem0, ssem1):
  cid = lax.axis_index("c")
  sid = lax.axis_index("s")
  wid = cid * NS + sid
  rows_per_s = NPAD // NS
  rows_per_w = NPAD // NW
  ed = (ed0, ed1, ed2, ed3)
  grows = (grow0, grow1)
  esems = (esem0, esem1)
  ssems = (ssem0, ssem1)

  pltpu.sync_copy(idx.at[pl.ds(wid * rows_per_w, rows_per_w)], x_v)
  pltpu.async_copy(table.at[x_v.at[pl.ds(0, CH)]], grow0, sem)
  pltpu.async_copy(table.at[x_v.at[pl.ds(CH, CH)]], grow1, sem)
  pltpu.make_async_copy(table.at[x_v.at[pl.ds(0, CH)]], grow0, sem).wait()
  pltpu.sync_copy(grow0, h0_out.at[pl.ds(wid * rows_per_w, CH)])
  pltpu.async_copy(table.at[x_v.at[pl.ds(2 * CH, 64)]],
                   grow0.at[pl.ds(0, 64)], sem)
  pltpu.make_async_copy(table.at[x_v.at[pl.ds(CH, CH)]], grow1, sem).wait()
  pltpu.sync_copy(grow1, h0_out.at[pl.ds(wid * rows_per_w + CH, CH)])
  pltpu.make_async_copy(table.at[x_v.at[pl.ds(2 * CH, 64)]],
                        grow0.at[pl.ds(0, 64)], sem).wait()
  pltpu.sync_copy(grow0.at[pl.ds(0, 64)],
                  h0_out.at[pl.ds(wid * rows_per_w + 2 * CH, 64)])

  _zero_vmem(grow0, CH, D)
  _zero_vmem(grow1, CH, D)
  for t in range(rows_per_s // CH):
    pltpu.sync_copy(grow0, accE.at[pl.ds(sid * rows_per_s + t * CH, CH)])
  plsc.subcore_barrier()

  pltpu.sync_copy(g_cm, g_v)
  riota = lax.iota(jnp.int32, L)

  def build(grow, edc):
    for j in range(CH // L):
      a16 = edc[2, pl.ds(j * L, L)]
      rr = riota + j * L
      for col in range(DE):
        val = plsc.load_gather(g_v, [a16 + col * NRELP])
        plsc.store_scatter(grow, [rr, jnp.full((L,), col, jnp.int32)], val)

  def idxload(c, slot, sem_):
    pltpu.async_copy(edata.at[pl.ds((wid * NCHW + c) * 8, 8)], ed[slot], sem_)

  def ewait(slot, sem_):
    pltpu.make_async_copy(edata.at[pl.ds(0, 8)], ed[slot], sem_).wait()

  def swait(grow, sem_):
    pltpu.make_async_copy(grow, accE.at[ed0.at[1]], sem_).wait()

  pltpu.sync_copy(edata.at[pl.ds((wid * NCHW) * 8, 8)], ed0)
  pltpu.sync_copy(edata.at[pl.ds((wid * NCHW + 1) * 8, 8)], ed1)

  def body(i, _):
    for k in range(4):
      c = 4 * i + k
      b = k % 2
      if k < 2:
        @pl.when(i > 0)
        def _():
          swait(grows[b], ssems[b])
          ewait(k, esems[b])
      else:
        swait(grows[b], ssems[b])
        ewait(k, esems[b])
      if k < 2:
        idxload(c + 2, (k + 2) % 4, esems[b])
      else:
        @pl.when(i < NCHW // 4 - 1)
        def _():
          idxload(c + 2, (k + 2) % 4, esems[b])
      build(grows[b], ed[k])
      pltpu.async_copy(grows[b], accE.at[ed[k].at[1]], ssems[b], add=True)
    return 0
  lax.fori_loop(0, NCHW // 4, body, 0)
  swait(grow0, ssem0)
  swait(grow1, ssem1)
  plsc.subcore_barrier()
  for t in range(rows_per_s // CH):
    r0 = sid * rows_per_s + t * CH
    pltpu.sync_copy(accE.at[pl.ds(r0, CH)], e_out.at[cid, pl.ds(r0, CH)])


def _k1(table, idx, edata, g_cm):
  return pl.kernel(
      _k1_body,
      out_type=(jax.ShapeDtypeStruct((NPAD, D), jnp.float32),
                jax.ShapeDtypeStruct((NC, NPAD, D), jnp.float32)),
      mesh=_mesh,
      scratch_types=[
          pltpu.VMEM_SHARED((NPAD, D), jnp.float32),
          pltpu.VMEM((NPAD // NW,), jnp.int32),
          pltpu.VMEM((DE * NRELP,), jnp.float32),
          pltpu.VMEM((CH, D), jnp.float32),
          pltpu.VMEM((CH, D), jnp.float32),
          pltpu.VMEM((8, CH), jnp.int32),
          pltpu.VMEM((8, CH), jnp.int32),
          pltpu.VMEM((8, CH), jnp.int32),
          pltpu.VMEM((8, CH), jnp.int32),
          pltpu.SemaphoreType.DMA,
          pltpu.SemaphoreType.DMA,
          pltpu.SemaphoreType.DMA,
          pltpu.SemaphoreType.DMA,
          pltpu.SemaphoreType.DMA,
      ],
      compiler_params=_sc_params,
  )(table, idx, edata, g_cm)


def _scale_rows(edc, rows, prior_v, prior_c):
  for j in range(CH // L):
    a16 = edc[2, pl.ds(j * L, L)]
    prior_c[pl.ds(j * L, L)] = plsc.load_gather(prior_v, [a16])

  @plsc.parallel_loop(0, CH, 1, unroll=4)
  def _(e):
    ev = jnp.full((L,), e, jnp.int32)
    pe = plsc.load_gather(prior_c, [ev])
    for k in range(D // L):
      rows[e, pl.ds(k * L, L)] = rows[e, pl.ds(k * L, L)] * pe


def _spmm_loop(h, edata, wid, ed, rows, prior_v, prior_c, accP, esems, gsems,
               ssems):
  def idxload(c, slot, sem_):
    pltpu.async_copy(edata.at[pl.ds((wid * NCHW + c) * 8, 8)], ed[slot], sem_)

  def ewait(slot, sem_):
    pltpu.make_async_copy(edata.at[pl.ds(0, 8)], ed[slot], sem_).wait()

  def gwait(rowsb, sem_):
    pltpu.make_async_copy(h.at[ed[0].at[0]], rowsb, sem_).wait()

  def swait(rowsb, sem_):
    pltpu.make_async_copy(rowsb, accP.at[ed[0].at[1]], sem_).wait()

  pltpu.sync_copy(edata.at[pl.ds((wid * NCHW) * 8, 8)], ed[0])
  pltpu.sync_copy(edata.at[pl.ds((wid * NCHW + 1) * 8, 8)], ed[1])
  pltpu.async_copy(h.at[ed[0].at[0]], rows[0], gsems[0])

  def body(i, _):
    for k in range(4):
      c = 4 * i + k
      b = k % 2
      nb = 1 - b
      if k == 0:
        @pl.when(i > 0)
        def _():
          swait(rows[nb], ssems[nb])
      else:
        swait(rows[nb], ssems[nb])
      if k < 3:
        if k >= 1:
          ewait((k + 1) % 4, esems[nb])
        else:
          @pl.when(i > 0)
          def _():
            ewait(1, esems[nb])
        pltpu.async_copy(h.at[ed[(k + 1) % 4].at[0]], rows[nb], gsems[nb])
      else:
        @pl.when(i < NCHW // 4 - 1)
        def _():
          ewait(0, esems[nb])
          pltpu.async_copy(h.at[ed[0].at[0]], rows[nb], gsems[nb])
      if k < 2:
        idxload(c + 2, (k + 2) % 4, esems[b])
      else:
        @pl.when(i < NCHW // 4 - 1)
        def _():
          idxload(c + 2, (k + 2) % 4, esems[b])
      gwait(rows[b], gsems[b])
      _scale_rows(ed[k], rows[b], prior_v, prior_c)
      pltpu.async_copy(rows[b], accP.at[ed[k].at[1]], ssems[b], add=True)
    return 0
  lax.fori_loop(0, NCHW // 4, body, 0)
  swait(rows[1], ssems[1])


_SPMM_SCRATCH = [
    pltpu.VMEM_SHARED((NPAD, D), jnp.float32),
    pltpu.VMEM((CH, D), jnp.float32),
    pltpu.VMEM((CH, D), jnp.float32),
    pltpu.VMEM((8, CH), jnp.int32),
    pltpu.VMEM((8, CH), jnp.int32),
    pltpu.VMEM((8, CH), jnp.int32),
    pltpu.VMEM((8, CH), jnp.int32),
    pltpu.VMEM((NRELP,), jnp.float32),
    pltpu.VMEM((CH,), jnp.float32),
    pltpu.SemaphoreType.DMA,
    pltpu.SemaphoreType.DMA,
    pltpu.SemaphoreType.DMA,
    pltpu.SemaphoreType.DMA,
    pltpu.SemaphoreType.DMA,
    pltpu.SemaphoreType.DMA,
]


def _spmm_prelude(accP, rows0, sid):
  rows_per_s = NPAD // NS
  _zero_vmem(rows0, CH, D)
  for t in range(rows_per_s // CH):
    pltpu.sync_copy(rows0, accP.at[pl.ds(sid * rows_per_s + t * CH, CH)])
  plsc.subcore_barrier()


def _spmm1_body(h, edata, prior_t,
                p_out,
                accP, rows0, rows1, ed0, ed1, ed2, ed3, prior_v, prior_c,
                esem0, esem1, gsem0, gsem1, ssem0, ssem1):
  cid = lax.axis_index("c")
  sid = lax.axis_index("s")
  wid = cid * NS + sid
  rows_per_s = NPAD // NS
  _spmm_prelude(accP, rows0, sid)
  pltpu.sync_copy(prior_t, prior_v)
  _spmm_loop(h, edata, wid, (ed0, ed1, ed2, ed3), (rows0, rows1),
             prior_v, prior_c, accP, (esem0, esem1), (gsem0, gsem1),
             (ssem0, ssem1))
  plsc.subcore_barrier()
  for t in range(rows_per_s // CH):
    r0 = sid * rows_per_s + t * CH
    pltpu.sync_copy(accP.at[pl.ds(r0, CH)], p_out.at[cid, pl.ds(r0, CH)])


def _spmm1(h, edata, prior_t):
  return pl.kernel(
      _spmm1_body,
      out_type=jax.ShapeDtypeStruct((NC, NPAD, D), jnp.float32),
      mesh=_mesh,
      scratch_types=list(_SPMM_SCRATCH),
      compiler_params=_sc_params,
  )(h, edata, prior_t)


def _spmm2_body(h, edata, prior_t, tix, t2,
                pg_out, hg_out, tg_out,
                accP, rows0, rows1, ed0, ed1, ed2, ed3, prior_v, prior_c,
                esem0, esem1, gsem0, gsem1, ssem0, ssem1,
                tbuf, tbuf2, sem2):
  cid = lax.axis_index("c")
  sid = lax.axis_index("s")
  wid = cid * NS + sid
  _spmm_prelude(accP, rows0, sid)
  pltpu.sync_copy(prior_t, prior_v)
  _spmm_loop(h, edata, wid, (ed0, ed1, ed2, ed3), (rows0, rows1),
             prior_v, prior_c, accP, (esem0, esem1), (gsem0, gsem1),
             (ssem0, ssem1))
  plsc.subcore_barrier()

  rb = BATCH // NS
  grows = rows0.at[pl.ds(0, rb)]
  pltpu.sync_copy(tix.at[pl.ds(sid * rb, rb)], tbuf)
  pltpu.async_copy(accP.at[tbuf], grows, sem2).wait()
  pltpu.sync_copy(grows, pg_out.at[cid, pl.ds(sid * rb, rb)])
  rb2 = BATCH // NW
  grows2 = rows1.at[pl.ds(0, rb2)]
  pltpu.sync_copy(tix.at[pl.ds(wid * rb2, rb2)], tbuf2)
  pltpu.async_copy(h.at[tbuf2], grows2, sem2).wait()
  pltpu.sync_copy(grows2, hg_out.at[pl.ds(wid * rb2, rb2)])
  pltpu.async_copy(t2.at[tbuf2], grows2, sem2).wait()
  pltpu.sync_copy(grows2, tg_out.at[pl.ds(wid * rb2, rb2)])


def _spmm2(h, edata, prior_t, tix, t2):
  return pl.kernel(
      _spmm2_body,
      out_type=(jax.ShapeDtypeStruct((NC, BATCH, D), jnp.float32),
                jax.ShapeDtypeStruct((BATCH, D), jnp.float32),
                jax.ShapeDtypeStruct((BATCH, D), jnp.float32)),
      mesh=_mesh,
      scratch_types=list(_SPMM_SCRATCH) + [
          pltpu.VMEM((BATCH // NS,), jnp.int32),
          pltpu.VMEM((BATCH // NW,), jnp.int32),
          pltpu.SemaphoreType.DMA,
      ],
      compiler_params=_sc_params,
  )(h, edata, prior_t, tix, t2)


_BLK = 1024


def _layer1_tc_body(p_ref, e_ref, h0_ref, ws_ref, we0_ref, we1_ref, wf_ref,
                    b0_ref, b1_ref, h1_ref, t2_ref):
  ps = p_ref[0] + p_ref[1]
  es = e_ref[0] + e_ref[1]
  acc = jnp.dot(ps, ws_ref[...], preferred_element_type=jnp.float32)
  acc += jnp.dot(es, we0_ref[...], preferred_element_type=jnp.float32)
  acc += jnp.dot(h0_ref[...], wf_ref[...], preferred_element_type=jnp.float32)
  h1_ref[...] = jnp.maximum(acc + b0_ref[...], 0.0)
  t2_ref[...] = jnp.dot(es, we1_ref[...],
                        preferred_element_type=jnp.float32) + b1_ref[...]


def _layer1_tc(p, e, h0, ws0, we0p, we1p, wf0, b0, b1):
  nblk = NPAD // _BLK
  return pl.pallas_call(
      _layer1_tc_body,
      grid=(nblk,),
      in_specs=[
          pl.BlockSpec((NC, _BLK, D), lambda i: (0, i, 0)),
          pl.BlockSpec((NC, _BLK, D), lambda i: (0, i, 0)),
          pl.BlockSpec((_BLK, D), lambda i: (i, 0)),
          pl.BlockSpec((D, D), lambda i: (0, 0)),
          pl.BlockSpec((D, D), lambda i: (0, 0)),
          pl.BlockSpec((D, D), lambda i: (0, 0)),
          pl.BlockSpec((D, D), lambda i: (0, 0)),
          pl.BlockSpec((1, D), lambda i: (0, 0)),
          pl.BlockSpec((1, D), lambda i: (0, 0)),
      ],
      out_specs=[
          pl.BlockSpec((_BLK, D), lambda i: (i, 0)),
          pl.BlockSpec((_BLK, D), lambda i: (i, 0)),
      ],
      out_shape=[
          jax.ShapeDtypeStruct((NPAD, D), jnp.float32),
          jax.ShapeDtypeStruct((NPAD, D), jnp.float32),
      ],
  )(p, e, h0, ws0, we0p, we1p, wf0, b0, b1)


def _final_tc_body(pg_ref, hg_ref, tg_ref, ws_ref, wf_ref, out_ref):
  ps = pg_ref[0] + pg_ref[1]
  acc = jnp.dot(ps, ws_ref[...], preferred_element_type=jnp.float32)
  acc += jnp.dot(hg_ref[...], wf_ref[...], preferred_element_type=jnp.float32)
  out_ref[...] = jnp.maximum(acc + tg_ref[...], 0.0)


def _final_tc(pg, hg, tg, ws1, wf1):
  return pl.pallas_call(
      _final_tc_body,
      out_shape=jax.ShapeDtypeStruct((BATCH, D), jnp.float32),
  )(pg, hg, tg, ws1, wf1)


def kernel(x, edge_index, edge_attr, y, num_size, entity_table, rel_table,
           rel_prior, W_src, W_self, W_edge, b):
  x = x.astype(jnp.int32)
  src = edge_index[0].astype(jnp.int32)
  dst = edge_index[1].astype(jnp.int32)
  attr = edge_attr.astype(jnp.int32)
  y = y.astype(jnp.int32)
  num_size = num_size.astype(jnp.int32)

  x_pad = jnp.pad(x, (0, NPAD - N_NODES))
  pad_n = EPAD - N_EDGES
  src_p = jnp.pad(src, (0, pad_n))
  dst_fill = (NPAD - CH) + jnp.arange(pad_n, dtype=jnp.int32) % CH
  dst_p = jnp.concatenate([dst, dst_fill])
  attr_p = jnp.pad(attr, (0, pad_n), constant_values=NRELP - L)
  nrel = rel_prior.shape[0]
  prior_flat = jnp.pad(rel_prior[:, 0], (0, NRELP - nrel))
  g = rel_table * rel_prior
  g_pad = jnp.pad(g, ((0, NRELP - nrel), (0, 0)))
  g_cm = g_pad.T.reshape(-1)

  ed3 = jnp.stack([src_p, dst_p, attr_p])
  edata = jnp.pad(ed3.reshape(3, NW, NCHW, CH).transpose(1, 2, 0, 3),
                  ((0, 0), (0, 0), (0, 5), (0, 0))
                  ).reshape(NW * NCHW * 8, CH)

  we0p = jnp.pad(W_edge[0], ((0, D - DE), (0, 0)))
  we1p = jnp.pad(W_edge[1], ((0, D - DE), (0, 0)))

  offsets = jnp.concatenate(
      [jnp.zeros((1,), jnp.int32), jnp.cumsum(num_size)[:-1]])
  true_idx = (offsets + y).astype(jnp.int32)

  b2 = b.reshape(2, 1, D)

  h0, eagg = _k1(entity_table, x_pad, edata, g_cm)
  p1 = _spmm1(h0, edata, prior_flat)
  h1, t2 = _layer1_tc(p1, eagg, h0, W_src[0], we0p, we1p,
                      W_self[0], b2[0], b2[1])
  pg, hg, tg = _spmm2(h1, edata, prior_flat, true_idx, t2)
  return _final_tc(pg, hg, tg, W_src[1], W_self[1])


# speedup vs baseline: 5.8997x; 1.0581x over previous
import jax
import jax.numpy as jnp
from jax import lax
from jax.experimental import pallas as pl
from jax.experimental.pallas import tpu as pltpu
from jax.experimental.pallas import tpu_sc as plsc

N_NODES = 10000
NPAD = 10240
N_EDGES = 320000
NC, NS, L = 2, 16, 16
NW = NC * NS
CH = 128
NCHW = 80
NCHW0, NCHW1 = 116, 44
EPAD = NW * NCHW * CH
D = 128
DE = 16
BATCH = 1024
NRELP = 256

_mesh = plsc.VectorSubcoreMesh(
    core_axis_name="c", subcore_axis_name="s", num_cores=NC, num_subcores=NS)
_sc_params = pltpu.CompilerParams(needs_layout_passes=False)


def _zero_vmem(ref, nrows, ncols):
  def body(r, _):
    for k in range(ncols // L):
      ref[r, pl.ds(k * L, L)] = jnp.zeros((L,), jnp.float32)
    return 0
  lax.fori_loop(0, nrows, body, 0)


def _k1_body(table, idx, edata, g_cm,
             h0_out, e_out,
             accE, x_v, g_v, grow0, grow1, ed0, ed1, ed2, ed3,
             sem, esem0, esem1, ssem0, ssem1):
  cid = lax.axis_index("c")
  sid = lax.axis_index("s")
  wid = cid * NS + sid
  rows_per_s = NPAD // NS
  rows_per_w = NPAD // NW
  ed = (ed0, ed1, ed2, ed3)
  grows = (grow0, grow1)
  esems = (esem0, esem1)
  ssems = (ssem0, ssem1)

  pltpu.sync_copy(idx.at[pl.ds(wid * rows_per_w, rows_per_w)], x_v)
  pltpu.async_copy(table.at[x_v.at[pl.ds(0, CH)]], grow0, sem)
  pltpu.async_copy(table.at[x_v.at[pl.ds(CH, CH)]], grow1, sem)
  pltpu.make_async_copy(table.at[x_v.at[pl.ds(0, CH)]], grow0, sem).wait()
  pltpu.sync_copy(grow0, h0_out.at[pl.ds(wid * rows_per_w, CH)])
  pltpu.async_copy(table.at[x_v.at[pl.ds(2 * CH, 64)]],
                   grow0.at[pl.ds(0, 64)], sem)
  pltpu.make_async_copy(table.at[x_v.at[pl.ds(CH, CH)]], grow1, sem).wait()
  pltpu.sync_copy(grow1, h0_out.at[pl.ds(wid * rows_per_w + CH, CH)])
  pltpu.make_async_copy(table.at[x_v.at[pl.ds(2 * CH, 64)]],
                        grow0.at[pl.ds(0, 64)], sem).wait()
  pltpu.sync_copy(grow0.at[pl.ds(0, 64)],
                  h0_out.at[pl.ds(wid * rows_per_w + 2 * CH, 64)])

  _zero_vmem(grow0, CH, D)
  _zero_vmem(grow1, CH, D)
  for t in range(rows_per_s // CH):
    pltpu.sync_copy(grow0, accE.at[pl.ds(sid * rows_per_s + t * CH, CH)])
  plsc.subcore_barrier()

  pltpu.sync_copy(g_cm, g_v)
  riota = lax.iota(jnp.int32, L)

  def build(grow, edc):
    for j in range(CH // L):
      a16 = edc[2, pl.ds(j * L, L)]
      rr = riota + j * L
      for col in range(DE):
        val = plsc.load_gather(g_v, [a16 + col * NRELP])
        plsc.store_scatter(grow, [rr, jnp.full((L,), col, jnp.int32)], val)

  def idxload(c, slot, sem_):
    pltpu.async_copy(edata.at[pl.ds((wid * NCHW + c) * 8, 8)], ed[slot], sem_)

  def ewait(slot, sem_):
    pltpu.make_async_copy(edata.at[pl.ds(0, 8)], ed[slot], sem_).wait()

  def swait(grow, sem_):
    pltpu.make_async_copy(grow, accE.at[ed0.at[1]], sem_).wait()

  pltpu.sync_copy(edata.at[pl.ds((wid * NCHW) * 8, 8)], ed0)
  pltpu.sync_copy(edata.at[pl.ds((wid * NCHW + 1) * 8, 8)], ed1)

  def body(i, _):
    for k in range(4):
      c = 4 * i + k
      b = k % 2
      if k < 2:
        @pl.when(i > 0)
        def _():
          swait(grows[b], ssems[b])
          ewait(k, esems[b])
      else:
        swait(grows[b], ssems[b])
        ewait(k, esems[b])
      if k < 2:
        idxload(c + 2, (k + 2) % 4, esems[b])
      else:
        @pl.when(i < NCHW // 4 - 1)
        def _():
          idxload(c + 2, (k + 2) % 4, esems[b])
      build(grows[b], ed[k])
      pltpu.async_copy(grows[b], accE.at[ed[k].at[1]], ssems[b], add=True)
    return 0
  lax.fori_loop(0, NCHW // 4, body, 0)
  swait(grow0, ssem0)
  swait(grow1, ssem1)
  plsc.subcore_barrier()
  for t in range(rows_per_s // CH):
    r0 = sid * rows_per_s + t * CH
    pltpu.sync_copy(accE.at[pl.ds(r0, CH)], e_out.at[cid, pl.ds(r0, CH)])


def _k1(table, idx, edata, g_cm):
  return pl.kernel(
      _k1_body,
      out_type=(jax.ShapeDtypeStruct((NPAD, D), jnp.float32),
                jax.ShapeDtypeStruct((NC, NPAD, D), jnp.float32)),
      mesh=_mesh,
      scratch_types=[
          pltpu.VMEM_SHARED((NPAD, D), jnp.float32),
          pltpu.VMEM((NPAD // NW,), jnp.int32),
          pltpu.VMEM((DE * NRELP,), jnp.float32),
          pltpu.VMEM((CH, D), jnp.float32),
          pltpu.VMEM((CH, D), jnp.float32),
          pltpu.VMEM((8, CH), jnp.int32),
          pltpu.VMEM((8, CH), jnp.int32),
          pltpu.VMEM((8, CH), jnp.int32),
          pltpu.VMEM((8, CH), jnp.int32),
          pltpu.SemaphoreType.DMA,
          pltpu.SemaphoreType.DMA,
          pltpu.SemaphoreType.DMA,
          pltpu.SemaphoreType.DMA,
          pltpu.SemaphoreType.DMA,
      ],
      compiler_params=_sc_params,
  )(table, idx, edata, g_cm)


def _scale_rows(edc, rows, prior_v, prior_c):
  for j in range(CH // L):
    a16 = edc[2, pl.ds(j * L, L)]
    prior_c[pl.ds(j * L, L)] = plsc.load_gather(prior_v, [a16])

  @plsc.parallel_loop(0, CH, 1, unroll=4)
  def _(e):
    ev = jnp.full((L,), e, jnp.int32)
    pe = plsc.load_gather(prior_c, [ev])
    for k in range(D // L):
      rows[e, pl.ds(k * L, L)] = rows[e, pl.ds(k * L, L)] * pe


def _spmm_loop(h, edata, base, nch4, ed, rows, prior_v, prior_c, accP, esems,
               gsems, ssems):
  def idxload(c, slot, sem_):
    pltpu.async_copy(edata.at[pl.ds((base + c) * 8, 8)], ed[slot], sem_)

  def ewait(slot, sem_):
    pltpu.make_async_copy(edata.at[pl.ds(0, 8)], ed[slot], sem_).wait()

  def gwait(rowsb, sem_):
    pltpu.make_async_copy(h.at[ed[0].at[0]], rowsb, sem_).wait()

  def swait(rowsb, sem_):
    pltpu.make_async_copy(rowsb, accP.at[ed[0].at[1]], sem_).wait()

  pltpu.sync_copy(edata.at[pl.ds(base * 8, 8)], ed[0])
  pltpu.sync_copy(edata.at[pl.ds((base + 1) * 8, 8)], ed[1])
  pltpu.async_copy(h.at[ed[0].at[0]], rows[0], gsems[0])

  def body(i, _):
    for k in range(4):
      c = 4 * i + k
      b = k % 2
      nb = 1 - b
      if k == 0:
        @pl.when(i > 0)
        def _():
          swait(rows[nb], ssems[nb])
      else:
        swait(rows[nb], ssems[nb])
      if k < 3:
        if k >= 1:
          ewait((k + 1) % 4, esems[nb])
        else:
          @pl.when(i > 0)
          def _():
            ewait(1, esems[nb])
        pltpu.async_copy(h.at[ed[(k + 1) % 4].at[0]], rows[nb], gsems[nb])
      else:
        @pl.when(i < nch4 - 1)
        def _():
          ewait(0, esems[nb])
          pltpu.async_copy(h.at[ed[0].at[0]], rows[nb], gsems[nb])
      if k < 2:
        idxload(c + 2, (k + 2) % 4, esems[b])
      else:
        @pl.when(i < nch4 - 1)
        def _():
          idxload(c + 2, (k + 2) % 4, esems[b])
      gwait(rows[b], gsems[b])
      _scale_rows(ed[k], rows[b], prior_v, prior_c)
      pltpu.async_copy(rows[b], accP.at[ed[k].at[1]], ssems[b], add=True)
    return 0
  lax.fori_loop(0, nch4, body, 0)
  swait(rows[1], ssems[1])


_SPMM_SCRATCH = [
    pltpu.VMEM_SHARED((NPAD, D), jnp.float32),
    pltpu.VMEM((CH, D), jnp.float32),
    pltpu.VMEM((CH, D), jnp.float32),
    pltpu.VMEM((8, CH), jnp.int32),
    pltpu.VMEM((8, CH), jnp.int32),
    pltpu.VMEM((8, CH), jnp.int32),
    pltpu.VMEM((8, CH), jnp.int32),
    pltpu.VMEM((NRELP,), jnp.float32),
    pltpu.VMEM((CH,), jnp.float32),
    pltpu.SemaphoreType.DMA,
    pltpu.SemaphoreType.DMA,
    pltpu.SemaphoreType.DMA,
    pltpu.SemaphoreType.DMA,
    pltpu.SemaphoreType.DMA,
    pltpu.SemaphoreType.DMA,
]


def _spmm_prelude(accP, rows0, sid):
  rows_per_s = NPAD // NS
  _zero_vmem(rows0, CH, D)
  for t in range(rows_per_s // CH):
    pltpu.sync_copy(rows0, accP.at[pl.ds(sid * rows_per_s + t * CH, CH)])
  plsc.subcore_barrier()


def _spmm1_body(h, edata, prior_t,
                p_out,
                accP, rows0, rows1, ed0, ed1, ed2, ed3, prior_v, prior_c,
                esem0, esem1, gsem0, gsem1, ssem0, ssem1):
  cid = lax.axis_index("c")
  sid = lax.axis_index("s")
  rows_per_s = NPAD // NS
  _spmm_prelude(accP, rows0, sid)
  pltpu.sync_copy(prior_t, prior_v)
  base = jnp.where(cid == 0, sid * NCHW0, NS * NCHW0 + sid * NCHW1)
  nch4 = jnp.where(cid == 0, NCHW0 // 4, NCHW1 // 4)
  _spmm_loop(h, edata, base, nch4, (ed0, ed1, ed2, ed3), (rows0, rows1),
             prior_v, prior_c, accP, (esem0, esem1), (gsem0, gsem1),
             (ssem0, ssem1))
  plsc.subcore_barrier()
  for t in range(rows_per_s // CH):
    r0 = sid * rows_per_s + t * CH
    pltpu.sync_copy(accP.at[pl.ds(r0, CH)], p_out.at[cid, pl.ds(r0, CH)])


def _spmm1(h, edata, prior_t):
  return pl.kernel(
      _spmm1_body,
      out_type=jax.ShapeDtypeStruct((NC, NPAD, D), jnp.float32),
      mesh=_mesh,
      scratch_types=list(_SPMM_SCRATCH),
      compiler_params=_sc_params,
  )(h, edata, prior_t)


def _spmm2_body(h, edata, prior_t, tix, t2,
                pg_out, hg_out, tg_out,
                accP, rows0, rows1, ed0, ed1, ed2, ed3, prior_v, prior_c,
                esem0, esem1, gsem0, gsem1, ssem0, ssem1,
                tbuf, tbuf2, sem2):
  cid = lax.axis_index("c")
  sid = lax.axis_index("s")
  wid = cid * NS + sid
  _spmm_prelude(accP, rows0, sid)
  pltpu.sync_copy(prior_t, prior_v)
  base = jnp.where(cid == 0, sid * NCHW0, NS * NCHW0 + sid * NCHW1)
  nch4 = jnp.where(cid == 0, NCHW0 // 4, NCHW1 // 4)
  _spmm_loop(h, edata, base, nch4, (ed0, ed1, ed2, ed3), (rows0, rows1),
             prior_v, prior_c, accP, (esem0, esem1), (gsem0, gsem1),
             (ssem0, ssem1))
  plsc.subcore_barrier()

  rb = BATCH // NS
  grows = rows0.at[pl.ds(0, rb)]
  pltpu.sync_copy(tix.at[pl.ds(sid * rb, rb)], tbuf)
  pltpu.async_copy(accP.at[tbuf], grows, sem2).wait()
  pltpu.sync_copy(grows, pg_out.at[cid, pl.ds(sid * rb, rb)])
  rb2 = BATCH // NW
  grows2 = rows1.at[pl.ds(0, rb2)]
  pltpu.sync_copy(tix.at[pl.ds(wid * rb2, rb2)], tbuf2)
  pltpu.async_copy(h.at[tbuf2], grows2, sem2).wait()
  pltpu.sync_copy(grows2, hg_out.at[pl.ds(wid * rb2, rb2)])
  pltpu.async_copy(t2.at[tbuf2], grows2, sem2).wait()
  pltpu.sync_copy(grows2, tg_out.at[pl.ds(wid * rb2, rb2)])


def _spmm2(h, edata, prior_t, tix, t2):
  return pl.kernel(
      _spmm2_body,
      out_type=(jax.ShapeDtypeStruct((NC, BATCH, D), jnp.float32),
                jax.ShapeDtypeStruct((BATCH, D), jnp.float32),
                jax.ShapeDtypeStruct((BATCH, D), jnp.float32)),
      mesh=_mesh,
      scratch_types=list(_SPMM_SCRATCH) + [
          pltpu.VMEM((BATCH // NS,), jnp.int32),
          pltpu.VMEM((BATCH // NW,), jnp.int32),
          pltpu.SemaphoreType.DMA,
      ],
      compiler_params=_sc_params,
  )(h, edata, prior_t, tix, t2)


_BLK = 1024


def _layer1_tc_body(p_ref, e_ref, h0_ref, ws_ref, we0_ref, we1_ref, wf_ref,
                    b0_ref, b1_ref, h1_ref, t2_ref):
  ps = p_ref[0] + p_ref[1]
  es = e_ref[0] + e_ref[1]
  acc = jnp.dot(ps, ws_ref[...], preferred_element_type=jnp.float32)
  acc += jnp.dot(es, we0_ref[...], preferred_element_type=jnp.float32)
  acc += jnp.dot(h0_ref[...], wf_ref[...], preferred_element_type=jnp.float32)
  h1_ref[...] = jnp.maximum(acc + b0_ref[...], 0.0)
  t2_ref[...] = jnp.dot(es, we1_ref[...],
                        preferred_element_type=jnp.float32) + b1_ref[...]


def _layer1_tc(p, e, h0, ws0, we0p, we1p, wf0, b0, b1):
  nblk = NPAD // _BLK
  return pl.pallas_call(
      _layer1_tc_body,
      grid=(nblk,),
      in_specs=[
          pl.BlockSpec((NC, _BLK, D), lambda i: (0, i, 0)),
          pl.BlockSpec((NC, _BLK, D), lambda i: (0, i, 0)),
          pl.BlockSpec((_BLK, D), lambda i: (i, 0)),
          pl.BlockSpec((D, D), lambda i: (0, 0)),
          pl.BlockSpec((D, D), lambda i: (0, 0)),
          pl.BlockSpec((D, D), lambda i: (0, 0)),
          pl.BlockSpec((D, D), lambda i: (0, 0)),
          pl.BlockSpec((1, D), lambda i: (0, 0)),
          pl.BlockSpec((1, D), lambda i: (0, 0)),
      ],
      out_specs=[
          pl.BlockSpec((_BLK, D), lambda i: (i, 0)),
          pl.BlockSpec((_BLK, D), lambda i: (i, 0)),
      ],
      out_shape=[
          jax.ShapeDtypeStruct((NPAD, D), jnp.float32),
          jax.ShapeDtypeStruct((NPAD, D), jnp.float32),
      ],
  )(p, e, h0, ws0, we0p, we1p, wf0, b0, b1)


def _final_tc_body(pg_ref, hg_ref, tg_ref, ws_ref, wf_ref, out_ref):
  ps = pg_ref[0] + pg_ref[1]
  acc = jnp.dot(ps, ws_ref[...], preferred_element_type=jnp.float32)
  acc += jnp.dot(hg_ref[...], wf_ref[...], preferred_element_type=jnp.float32)
  out_ref[...] = jnp.maximum(acc + tg_ref[...], 0.0)


def _final_tc(pg, hg, tg, ws1, wf1):
  return pl.pallas_call(
      _final_tc_body,
      out_shape=jax.ShapeDtypeStruct((BATCH, D), jnp.float32),
  )(pg, hg, tg, ws1, wf1)


def kernel(x, edge_index, edge_attr, y, num_size, entity_table, rel_table,
           rel_prior, W_src, W_self, W_edge, b):
  x = x.astype(jnp.int32)
  src = edge_index[0].astype(jnp.int32)
  dst = edge_index[1].astype(jnp.int32)
  attr = edge_attr.astype(jnp.int32)
  y = y.astype(jnp.int32)
  num_size = num_size.astype(jnp.int32)

  x_pad = jnp.pad(x, (0, NPAD - N_NODES))
  pad_n = EPAD - N_EDGES
  src_p = jnp.pad(src, (0, pad_n))
  dst_fill = (NPAD - CH) + jnp.arange(pad_n, dtype=jnp.int32) % CH
  dst_p = jnp.concatenate([dst, dst_fill])
  attr_p = jnp.pad(attr, (0, pad_n), constant_values=NRELP - L)
  nrel = rel_prior.shape[0]
  prior_flat = jnp.pad(rel_prior[:, 0], (0, NRELP - nrel))
  g = rel_table * rel_prior
  g_pad = jnp.pad(g, ((0, NRELP - nrel), (0, 0)))
  g_cm = g_pad.T.reshape(-1)

  ed3 = jnp.stack([src_p, dst_p, attr_p])
  edata = jnp.pad(ed3.reshape(3, NW, NCHW, CH).transpose(1, 2, 0, 3),
                  ((0, 0), (0, 0), (0, 5), (0, 0))
                  ).reshape(NW * NCHW * 8, CH)

  we0p = jnp.pad(W_edge[0], ((0, D - DE), (0, 0)))
  we1p = jnp.pad(W_edge[1], ((0, D - DE), (0, 0)))

  offsets = jnp.concatenate(
      [jnp.zeros((1,), jnp.int32), jnp.cumsum(num_size)[:-1]])
  true_idx = (offsets + y).astype(jnp.int32)

  b2 = b.reshape(2, 1, D)

  h0, eagg = _k1(entity_table, x_pad, edata, g_cm)
  p1 = _spmm1(h0, edata, prior_flat)
  h1, t2 = _layer1_tc(p1, eagg, h0, W_src[0], we0p, we1p,
                      W_self[0], b2[0], b2[1])
  pg, hg, tg = _spmm2(h1, edata, prior_flat, true_idx, t2)
  return _final_tc(pg, hg, tg, W_src[1], W_self[1])


# speedup vs baseline: 5.9025x; 1.0005x over previous
import jax
import jax.numpy as jnp
from jax import lax
from jax.experimental import pallas as pl
from jax.experimental.pallas import tpu as pltpu
from jax.experimental.pallas import tpu_sc as plsc

N_NODES = 10000
NPAD = 10240
N_EDGES = 320000
NC, NS, L = 2, 16, 16
NW = NC * NS
CH = 128
NCHW = 80
NCHW0, NCHW1 = 116, 44
EPAD = NW * NCHW * CH
D = 128
DE = 16
BATCH = 1024
NRELP = 256

_sc_params = pltpu.CompilerParams(needs_layout_passes=False)
_mesh_cache = []


def _mesh():
  if not _mesh_cache:
    _mesh_cache.append(plsc.VectorSubcoreMesh(
        core_axis_name="c", subcore_axis_name="s",
        num_cores=NC, num_subcores=NS))
  return _mesh_cache[0]


def _zero_vmem(ref, nrows, ncols):
  def body(r, _):
    for k in range(ncols // L):
      ref[r, pl.ds(k * L, L)] = jnp.zeros((L,), jnp.float32)
    return 0
  lax.fori_loop(0, nrows, body, 0)


def _k1_body(table, idx, edata, g_cm,
             h0_out, e_out,
             accE, x_v, g_v, grow0, grow1, ed0, ed1, ed2, ed3,
             sem, esem0, esem1, ssem0, ssem1):
  cid = lax.axis_index("c")
  sid = lax.axis_index("s")
  wid = cid * NS + sid
  rows_per_s = NPAD // NS
  rows_per_w = NPAD // NW
  ed = (ed0, ed1, ed2, ed3)
  grows = (grow0, grow1)
  esems = (esem0, esem1)
  ssems = (ssem0, ssem1)

  pltpu.sync_copy(idx.at[pl.ds(wid * rows_per_w, rows_per_w)], x_v)
  pltpu.async_copy(table.at[x_v.at[pl.ds(0, CH)]], grow0, sem)
  pltpu.async_copy(table.at[x_v.at[pl.ds(CH, CH)]], grow1, sem)
  pltpu.make_async_copy(table.at[x_v.at[pl.ds(0, CH)]], grow0, sem).wait()
  pltpu.sync_copy(grow0, h0_out.at[pl.ds(wid * rows_per_w, CH)])
  pltpu.async_copy(table.at[x_v.at[pl.ds(2 * CH, 64)]],
                   grow0.at[pl.ds(0, 64)], sem)
  pltpu.make_async_copy(table.at[x_v.at[pl.ds(CH, CH)]], grow1, sem).wait()
  pltpu.sync_copy(grow1, h0_out.at[pl.ds(wid * rows_per_w + CH, CH)])
  pltpu.make_async_copy(table.at[x_v.at[pl.ds(2 * CH, 64)]],
                        grow0.at[pl.ds(0, 64)], sem).wait()
  pltpu.sync_copy(grow0.at[pl.ds(0, 64)],
                  h0_out.at[pl.ds(wid * rows_per_w + 2 * CH, 64)])

  _zero_vmem(grow0, CH, D)
  _zero_vmem(grow1, CH, D)
  for t in range(rows_per_s // CH):
    pltpu.sync_copy(grow0, accE.at[pl.ds(sid * rows_per_s + t * CH, CH)])
  plsc.subcore_barrier()

  pltpu.sync_copy(g_cm, g_v)
  riota = lax.iota(jnp.int32, L)

  def build(grow, edc):
    for j in range(CH // L):
      a16 = edc[2, pl.ds(j * L, L)]
      rr = riota + j * L
      for col in range(DE):
        val = plsc.load_gather(g_v, [a16 + col * NRELP])
        plsc.store_scatter(grow, [rr, jnp.full((L,), col, jnp.int32)], val)

  def idxload(c, slot, sem_):
    pltpu.async_copy(edata.at[pl.ds((wid * NCHW + c) * 8, 8)], ed[slot], sem_)

  def ewait(slot, sem_):
    pltpu.make_async_copy(edata.at[pl.ds(0, 8)], ed[slot], sem_).wait()

  def swait(grow, sem_):
    pltpu.make_async_copy(grow, accE.at[ed0.at[1]], sem_).wait()

  pltpu.sync_copy(edata.at[pl.ds((wid * NCHW) * 8, 8)], ed0)
  pltpu.sync_copy(edata.at[pl.ds((wid * NCHW + 1) * 8, 8)], ed1)

  def body(i, _):
    for k in range(4):
      c = 4 * i + k
      b = k % 2
      if k < 2:
        @pl.when(i > 0)
        def _():
          swait(grows[b], ssems[b])
          ewait(k, esems[b])
      else:
        swait(grows[b], ssems[b])
        ewait(k, esems[b])
      if k < 2:
        idxload(c + 2, (k + 2) % 4, esems[b])
      else:
        @pl.when(i < NCHW // 4 - 1)
        def _():
          idxload(c + 2, (k + 2) % 4, esems[b])
      build(grows[b], ed[k])
      pltpu.async_copy(grows[b], accE.at[ed[k].at[1]], ssems[b], add=True)
    return 0
  lax.fori_loop(0, NCHW // 4, body, 0)
  swait(grow0, ssem0)
  swait(grow1, ssem1)
  plsc.subcore_barrier()
  for t in range(rows_per_s // CH):
    r0 = sid * rows_per_s + t * CH
    pltpu.sync_copy(accE.at[pl.ds(r0, CH)], e_out.at[cid, pl.ds(r0, CH)])


def _k1(table, idx, edata, g_cm):
  return pl.kernel(
      _k1_body,
      out_type=(jax.ShapeDtypeStruct((NPAD, D), jnp.float32),
                jax.ShapeDtypeStruct((NC, NPAD, D), jnp.float32)),
      mesh=_mesh(),
      scratch_types=[
          pltpu.VMEM_SHARED((NPAD, D), jnp.float32),
          pltpu.VMEM((NPAD // NW,), jnp.int32),
          pltpu.VMEM((DE * NRELP,), jnp.float32),
          pltpu.VMEM((CH, D), jnp.float32),
          pltpu.VMEM((CH, D), jnp.float32),
          pltpu.VMEM((8, CH), jnp.int32),
          pltpu.VMEM((8, CH), jnp.int32),
          pltpu.VMEM((8, CH), jnp.int32),
          pltpu.VMEM((8, CH), jnp.int32),
          pltpu.SemaphoreType.DMA,
          pltpu.SemaphoreType.DMA,
          pltpu.SemaphoreType.DMA,
          pltpu.SemaphoreType.DMA,
          pltpu.SemaphoreType.DMA,
      ],
      compiler_params=_sc_params,
  )(table, idx, edata, g_cm)


def _scale_rows(edc, rows, prior_v, prior_c):
  for j in range(CH // L):
    a16 = edc[2, pl.ds(j * L, L)]
    prior_c[pl.ds(j * L, L)] = plsc.load_gather(prior_v, [a16])

  @plsc.parallel_loop(0, CH, 1, unroll=4)
  def _(e):
    ev = jnp.full((L,), e, jnp.int32)
    pe = plsc.load_gather(prior_c, [ev])
    for k in range(D // L):
      rows[e, pl.ds(k * L, L)] = rows[e, pl.ds(k * L, L)] * pe


def _spmm_loop(h, edata, base, nch4, ed, rows, prior_v, prior_c, accP, esems,
               gsems, ssems):
  def idxload(c, slot, sem_):
    pltpu.async_copy(edata.at[pl.ds((base + c) * 8, 8)], ed[slot], sem_)

  def ewait(slot, sem_):
    pltpu.make_async_copy(edata.at[pl.ds(0, 8)], ed[slot], sem_).wait()

  def gwait(rowsb, sem_):
    pltpu.make_async_copy(h.at[ed[0].at[0]], rowsb, sem_).wait()

  def swait(rowsb, sem_):
    pltpu.make_async_copy(rowsb, accP.at[ed[0].at[1]], sem_).wait()

  pltpu.sync_copy(edata.at[pl.ds(base * 8, 8)], ed[0])
  pltpu.sync_copy(edata.at[pl.ds((base + 1) * 8, 8)], ed[1])
  pltpu.async_copy(h.at[ed[0].at[0]], rows[0], gsems[0])

  def body(i, _):
    for k in range(4):
      c = 4 * i + k
      b = k % 2
      nb = 1 - b
      if k == 0:
        @pl.when(i > 0)
        def _():
          swait(rows[nb], ssems[nb])
      else:
        swait(rows[nb], ssems[nb])
      if k < 3:
        if k >= 1:
          ewait((k + 1) % 4, esems[nb])
        else:
          @pl.when(i > 0)
          def _():
            ewait(1, esems[nb])
        pltpu.async_copy(h.at[ed[(k + 1) % 4].at[0]], rows[nb], gsems[nb])
      else:
        @pl.when(i < nch4 - 1)
        def _():
          ewait(0, esems[nb])
          pltpu.async_copy(h.at[ed[0].at[0]], rows[nb], gsems[nb])
      if k < 2:
        idxload(c + 2, (k + 2) % 4, esems[b])
      else:
        @pl.when(i < nch4 - 1)
        def _():
          idxload(c + 2, (k + 2) % 4, esems[b])
      gwait(rows[b], gsems[b])
      _scale_rows(ed[k], rows[b], prior_v, prior_c)
      pltpu.async_copy(rows[b], accP.at[ed[k].at[1]], ssems[b], add=True)
    return 0
  lax.fori_loop(0, nch4, body, 0)
  swait(rows[1], ssems[1])


_SPMM_SCRATCH = [
    pltpu.VMEM_SHARED((NPAD, D), jnp.float32),
    pltpu.VMEM((CH, D), jnp.float32),
    pltpu.VMEM((CH, D), jnp.float32),
    pltpu.VMEM((8, CH), jnp.int32),
    pltpu.VMEM((8, CH), jnp.int32),
    pltpu.VMEM((8, CH), jnp.int32),
    pltpu.VMEM((8, CH), jnp.int32),
    pltpu.VMEM((NRELP,), jnp.float32),
    pltpu.VMEM((CH,), jnp.float32),
    pltpu.SemaphoreType.DMA,
    pltpu.SemaphoreType.DMA,
    pltpu.SemaphoreType.DMA,
    pltpu.SemaphoreType.DMA,
    pltpu.SemaphoreType.DMA,
    pltpu.SemaphoreType.DMA,
]


def _spmm_prelude(accP, rows0, sid):
  rows_per_s = NPAD // NS
  _zero_vmem(rows0, CH, D)
  for t in range(rows_per_s // CH):
    pltpu.sync_copy(rows0, accP.at[pl.ds(sid * rows_per_s + t * CH, CH)])
  plsc.subcore_barrier()


def _spmm1_body(h, edata, prior_t,
                p_out,
                accP, rows0, rows1, ed0, ed1, ed2, ed3, prior_v, prior_c,
                esem0, esem1, gsem0, gsem1, ssem0, ssem1):
  cid = lax.axis_index("c")
  sid = lax.axis_index("s")
  rows_per_s = NPAD // NS
  _spmm_prelude(accP, rows0, sid)
  pltpu.sync_copy(prior_t, prior_v)
  base = jnp.where(cid == 0, sid * NCHW0, NS * NCHW0 + sid * NCHW1)
  nch4 = jnp.where(cid == 0, NCHW0 // 4, NCHW1 // 4)
  _spmm_loop(h, edata, base, nch4, (ed0, ed1, ed2, ed3), (rows0, rows1),
             prior_v, prior_c, accP, (esem0, esem1), (gsem0, gsem1),
             (ssem0, ssem1))
  plsc.subcore_barrier()
  for t in range(rows_per_s // CH):
    r0 = sid * rows_per_s + t * CH
    pltpu.sync_copy(accP.at[pl.ds(r0, CH)], p_out.at[cid, pl.ds(r0, CH)])


def _spmm1(h, edata, prior_t):
  return pl.kernel(
      _spmm1_body,
      out_type=jax.ShapeDtypeStruct((NC, NPAD, D), jnp.float32),
      mesh=_mesh(),
      scratch_types=list(_SPMM_SCRATCH),
      compiler_params=_sc_params,
  )(h, edata, prior_t)


def _spmm2_body(h, edata, prior_t, tix, t2,
                pg_out, hg_out, tg_out,
                accP, rows0, rows1, ed0, ed1, ed2, ed3, prior_v, prior_c,
                esem0, esem1, gsem0, gsem1, ssem0, ssem1,
                tbuf, tbuf2, sem2):
  cid = lax.axis_index("c")
  sid = lax.axis_index("s")
  wid = cid * NS + sid
  _spmm_prelude(accP, rows0, sid)
  pltpu.sync_copy(prior_t, prior_v)
  base = jnp.where(cid == 0, sid * NCHW0, NS * NCHW0 + sid * NCHW1)
  nch4 = jnp.where(cid == 0, NCHW0 // 4, NCHW1 // 4)
  _spmm_loop(h, edata, base, nch4, (ed0, ed1, ed2, ed3), (rows0, rows1),
             prior_v, prior_c, accP, (esem0, esem1), (gsem0, gsem1),
             (ssem0, ssem1))
  plsc.subcore_barrier()

  rb = BATCH // NS
  grows = rows0.at[pl.ds(0, rb)]
  pltpu.sync_copy(tix.at[pl.ds(sid * rb, rb)], tbuf)
  pltpu.async_copy(accP.at[tbuf], grows, sem2).wait()
  pltpu.sync_copy(grows, pg_out.at[cid, pl.ds(sid * rb, rb)])
  rb2 = BATCH // NW
  grows2 = rows1.at[pl.ds(0, rb2)]
  pltpu.sync_copy(tix.at[pl.ds(wid * rb2, rb2)], tbuf2)
  pltpu.async_copy(h.at[tbuf2], grows2, sem2).wait()
  pltpu.sync_copy(grows2, hg_out.at[pl.ds(wid * rb2, rb2)])
  pltpu.async_copy(t2.at[tbuf2], grows2, sem2).wait()
  pltpu.sync_copy(grows2, tg_out.at[pl.ds(wid * rb2, rb2)])


def _spmm2(h, edata, prior_t, tix, t2):
  return pl.kernel(
      _spmm2_body,
      out_type=(jax.ShapeDtypeStruct((NC, BATCH, D), jnp.float32),
                jax.ShapeDtypeStruct((BATCH, D), jnp.float32),
                jax.ShapeDtypeStruct((BATCH, D), jnp.float32)),
      mesh=_mesh(),
      scratch_types=list(_SPMM_SCRATCH) + [
          pltpu.VMEM((BATCH // NS,), jnp.int32),
          pltpu.VMEM((BATCH // NW,), jnp.int32),
          pltpu.SemaphoreType.DMA,
      ],
      compiler_params=_sc_params,
  )(h, edata, prior_t, tix, t2)


_BLK = 1024


def _layer1_tc_body(p_ref, e_ref, h0_ref, ws_ref, we0_ref, we1_ref, wf_ref,
                    b0_ref, b1_ref, h1_ref, t2_ref):
  ps = p_ref[0] + p_ref[1]
  es = e_ref[0] + e_ref[1]
  acc = jnp.dot(ps, ws_ref[...], preferred_element_type=jnp.float32)
  acc += jnp.dot(es, we0_ref[...], preferred_element_type=jnp.float32)
  acc += jnp.dot(h0_ref[...], wf_ref[...], preferred_element_type=jnp.float32)
  h1_ref[...] = jnp.maximum(acc + b0_ref[...], 0.0)
  t2_ref[...] = jnp.dot(es, we1_ref[...],
                        preferred_element_type=jnp.float32) + b1_ref[...]


def _layer1_tc(p, e, h0, ws0, we0p, we1p, wf0, b0, b1):
  nblk = NPAD // _BLK
  return pl.pallas_call(
      _layer1_tc_body,
      grid=(nblk,),
      in_specs=[
          pl.BlockSpec((NC, _BLK, D), lambda i: (0, i, 0)),
          pl.BlockSpec((NC, _BLK, D), lambda i: (0, i, 0)),
          pl.BlockSpec((_BLK, D), lambda i: (i, 0)),
          pl.BlockSpec((D, D), lambda i: (0, 0)),
          pl.BlockSpec((D, D), lambda i: (0, 0)),
          pl.BlockSpec((D, D), lambda i: (0, 0)),
          pl.BlockSpec((D, D), lambda i: (0, 0)),
          pl.BlockSpec((1, D), lambda i: (0, 0)),
          pl.BlockSpec((1, D), lambda i: (0, 0)),
      ],
      out_specs=[
          pl.BlockSpec((_BLK, D), lambda i: (i, 0)),
          pl.BlockSpec((_BLK, D), lambda i: (i, 0)),
      ],
      out_shape=[
          jax.ShapeDtypeStruct((NPAD, D), jnp.float32),
          jax.ShapeDtypeStruct((NPAD, D), jnp.float32),
      ],
  )(p, e, h0, ws0, we0p, we1p, wf0, b0, b1)


def _final_tc_body(pg_ref, hg_ref, tg_ref, ws_ref, wf_ref, out_ref):
  ps = pg_ref[0] + pg_ref[1]
  acc = jnp.dot(ps, ws_ref[...], preferred_element_type=jnp.float32)
  acc += jnp.dot(hg_ref[...], wf_ref[...], preferred_element_type=jnp.float32)
  out_ref[...] = jnp.maximum(acc + tg_ref[...], 0.0)


def _final_tc(pg, hg, tg, ws1, wf1):
  return pl.pallas_call(
      _final_tc_body,
      out_shape=jax.ShapeDtypeStruct((BATCH, D), jnp.float32),
  )(pg, hg, tg, ws1, wf1)


def kernel(x, edge_index, edge_attr, y, num_size, entity_table, rel_table,
           rel_prior, W_src, W_self, W_edge, b):
  x = x.astype(jnp.int32)
  src = edge_index[0].astype(jnp.int32)
  dst = edge_index[1].astype(jnp.int32)
  attr = edge_attr.astype(jnp.int32)
  y = y.astype(jnp.int32)
  num_size = num_size.astype(jnp.int32)

  x_pad = jnp.pad(x, (0, NPAD - N_NODES))
  pad_n = EPAD - N_EDGES
  src_p = jnp.pad(src, (0, pad_n))
  dst_fill = (NPAD - CH) + jnp.arange(pad_n, dtype=jnp.int32) % CH
  dst_p = jnp.concatenate([dst, dst_fill])
  attr_p = jnp.pad(attr, (0, pad_n), constant_values=NRELP - L)
  nrel = rel_prior.shape[0]
  prior_flat = jnp.pad(rel_prior[:, 0], (0, NRELP - nrel))
  g = rel_table * rel_prior
  g_pad = jnp.pad(g, ((0, NRELP - nrel), (0, 0)))
  g_cm = g_pad.T.reshape(-1)

  ed3 = jnp.stack([src_p, dst_p, attr_p])
  edata = jnp.pad(ed3.reshape(3, NW, NCHW, CH).transpose(1, 2, 0, 3),
                  ((0, 0), (0, 0), (0, 5), (0, 0))
                  ).reshape(NW * NCHW * 8, CH)

  we0p = jnp.pad(W_edge[0], ((0, D - DE), (0, 0)))
  we1p = jnp.pad(W_edge[1], ((0, D - DE), (0, 0)))

  offsets = jnp.concatenate(
      [jnp.zeros((1,), jnp.int32), jnp.cumsum(num_size)[:-1]])
  true_idx = (offsets + y).astype(jnp.int32)

  b2 = b.reshape(2, 1, D)

  h0, eagg = _k1(entity_table, x_pad, edata, g_cm)
  p1 = _spmm1(h0, edata, prior_flat)
  h1, t2 = _layer1_tc(p1, eagg, h0, W_src[0], we0p, we1p,
                      W_self[0], b2[0], b2[1])
  pg, hg, tg = _spmm2(h1, edata, prior_flat, true_idx, t2)
  return _final_tc(pg, hg, tg, W_src[1], W_self[1])
